# baseline (speedup 1.0000x reference)
import functools

import jax
import jax.numpy as jnp
from jax.experimental import pallas as pl
from jax.experimental.pallas import tpu as pltpu

P = 512
BLK = 2048
NEG_INF = float("-inf")


def _featurize_body(n, x_ref, wcat_ref, bcat_ref, of_ref, of2_ref, ox_ref):
    i = pl.program_id(0)
    xb = x_ref[...]
    y = jax.lax.dot_general(
        xb, wcat_ref[...], (((1,), (0,)), ((), ())),
        preferred_element_type=jnp.float32,
    ) + bcat_ref[...]
    ox = y[:, :8]
    logit = y[:, 8:9]
    of = jax.nn.sigmoid(logit)
    gid = i * BLK + jax.lax.broadcasted_iota(jnp.int32, (BLK, 1), 0)
    ofm = jnp.where(gid < n, of, NEG_INF)
    of_ref[...] = ofm
    of2_ref[...] = ofm.reshape(BLK // 128, 128)
    ox_ref[...] = ox


def _select_body(nrows, of2_ref, ox_ref, xp_ref, scr_ref):
    scr_ref[...] = of2_ref[...]
    lane = jax.lax.broadcasted_iota(jnp.int32, (1, 128), 1)
    lin = (jax.lax.broadcasted_iota(jnp.int32, (nrows, 128), 0) * 128
           + jax.lax.broadcasted_iota(jnp.int32, (nrows, 128), 1))

    def body(k, carry):
        arr = scr_ref[...]
        m = jnp.max(arr)
        idx = jnp.min(jnp.where(arr == m, lin, jnp.int32(2**31 - 1)))
        r = idx // 128
        c = idx - r * 128
        row = scr_ref[pl.ds(r, 1), :]
        scr_ref[pl.ds(r, 1), :] = jnp.where(lane == c, NEG_INF, row)
        xp_ref[pl.ds(k, 1), :] = ox_ref[pl.ds(idx, 1), :]
        return carry

    jax.lax.fori_loop(0, P, body, 0)


def _assign_body(ox_ref, xpt_ref, md_ref, inst_ref):
    oxb = ox_ref[...]
    xpt = xpt_ref[...]
    cross = jax.lax.dot_general(
        oxb, xpt, (((1,), (0,)), ((), ())),
        preferred_element_type=jnp.float32,
    )
    on = jnp.sum(oxb * oxb, axis=1, keepdims=True)
    pn = jnp.sum(xpt * xpt, axis=0, keepdims=True)
    dist = on + pn - 2.0 * cross
    md = jnp.min(dist, axis=1, keepdims=True)
    pid = jax.lax.broadcasted_iota(jnp.int32, (BLK, P), 1)
    nearest = jnp.min(jnp.where(dist == md, pid, P), axis=1, keepdims=True)
    md_ref[...] = md
    inst_ref[...] = jnp.where(md < 1.0, nearest, -1)


def kernel(x, W_beta, b_beta, W_coord, b_coord):
    n, f = x.shape
    d = W_coord.shape[1]
    nblk = (n + BLK - 1) // BLK
    nrows = nblk * BLK // 128

    wcat = jnp.concatenate([W_coord, W_beta], axis=1)
    bcat = jnp.concatenate([b_coord, b_beta])[None, :]

    of2d, of2, ox = pl.pallas_call(
        functools.partial(_featurize_body, n),
        grid=(nblk,),
        in_specs=[
            pl.BlockSpec((BLK, f), lambda i: (i, 0)),
            pl.BlockSpec((f, d + 1), lambda i: (0, 0)),
            pl.BlockSpec((1, d + 1), lambda i: (0, 0)),
        ],
        out_specs=[
            pl.BlockSpec((BLK, 1), lambda i: (i, 0)),
            pl.BlockSpec((BLK // 128, 128), lambda i: (i, 0)),
            pl.BlockSpec((BLK, d), lambda i: (i, 0)),
        ],
        out_shape=[
            jax.ShapeDtypeStruct((n, 1), jnp.float32),
            jax.ShapeDtypeStruct((nrows, 128), jnp.float32),
            jax.ShapeDtypeStruct((n, d), jnp.float32),
        ],
    )(x, wcat, bcat)

    xp = pl.pallas_call(
        functools.partial(_select_body, nrows),
        in_specs=[
            pl.BlockSpec((nrows, 128), lambda: (0, 0)),
            pl.BlockSpec((n, d), lambda: (0, 0)),
        ],
        out_specs=pl.BlockSpec((P, d), lambda: (0, 0)),
        out_shape=jax.ShapeDtypeStruct((P, d), jnp.float32),
        scratch_shapes=[pltpu.VMEM((nrows, 128), jnp.float32)],
    )(of2, ox)

    xpt = xp.T

    md2d, inst2d = pl.pallas_call(
        _assign_body,
        grid=(nblk,),
        in_specs=[
            pl.BlockSpec((BLK, d), lambda i: (i, 0)),
            pl.BlockSpec((d, P), lambda i: (0, 0)),
        ],
        out_specs=[
            pl.BlockSpec((BLK, 1), lambda i: (i, 0)),
            pl.BlockSpec((BLK, 1), lambda i: (i, 0)),
        ],
        out_shape=[
            jax.ShapeDtypeStruct((n, 1), jnp.float32),
            jax.ShapeDtypeStruct((n, 1), jnp.int32),
        ],
    )(ox, xpt)

    return (of2d.reshape(n), ox, md2d.reshape(n), inst2d.reshape(n))
